# Initial kernel scaffold; baseline (speedup 1.0000x reference)
import jax
import jax.numpy as jnp
from jax.experimental import pallas as pl


def kernel(input, W_orig, W_prompt):
    raise NotImplementedError("write your pallas kernel here")


# speedup vs baseline: 1.0241x; 1.0241x over previous
import functools

import jax
import jax.numpy as jnp
from jax import lax
from jax.experimental import pallas as pl
from jax.experimental.pallas import tpu as pltpu
from jax.experimental.pallas import tpu_sc as plsc

P = 64
B = 4096
L = 200
D = 64

NC = 2
NS = 16
NW = NC * NS


def kernel(input, W_orig, W_prompt):
    rows_per_w = B // NW
    mesh = plsc.VectorSubcoreMesh(core_axis_name="c", subcore_axis_name="s")

    @functools.partial(
        pl.kernel,
        mesh=mesh,
        out_type=jax.ShapeDtypeStruct((B, L, D), jnp.float32),
        compiler_params=pltpu.CompilerParams(use_tc_tiling_on_sc=False),
        scratch_types=[
            pltpu.VMEM((L,), jnp.int32),
            pltpu.VMEM((L, D), jnp.float32),
            pltpu.SemaphoreType.DMA,
        ],
    )
    def k(inp_hbm, worig_hbm, wprompt_hbm, out_hbm, idx_v, rows_v, sem):
        wid = lax.axis_index("s") * NC + lax.axis_index("c")
        base = wid * rows_per_w

        def body(i, carry):
            b = base + i
            pltpu.sync_copy(inp_hbm.at[b], idx_v)
            cp0 = pltpu.async_copy(
                wprompt_hbm.at[idx_v.at[pl.ds(0, 64)]],
                rows_v.at[pl.ds(0, 64)], sem)
            cp1 = pltpu.async_copy(
                worig_hbm.at[idx_v.at[pl.ds(64, 64)]],
                rows_v.at[pl.ds(64, 64)], sem)
            cp2 = pltpu.async_copy(
                worig_hbm.at[idx_v.at[pl.ds(128, 72)]],
                rows_v.at[pl.ds(128, 72)], sem)
            cp0.wait()
            cp1.wait()
            cp2.wait()
            pltpu.sync_copy(rows_v, out_hbm.at[b])
            return carry

        lax.fori_loop(0, rows_per_w, body, 0)

    return k(input, W_orig, W_prompt)
